# Initial kernel scaffold; baseline (speedup 1.0000x reference)
import jax
import jax.numpy as jnp
from jax.experimental import pallas as pl


def kernel(embed_sim, indexes, labels):
    raise NotImplementedError("write your pallas kernel here")


# speedup vs baseline: 10.8220x; 10.8220x over previous
import functools

import jax
import jax.numpy as jnp
from jax import lax
from jax.experimental import pallas as pl
from jax.experimental.pallas import tpu as pltpu
from jax.experimental.pallas import tpu_sc as plsc

_C = 80
_CP = 128
_BR = 512
_BC = 2048


def _gather_rows_sc(labels, indexes):
    _, d = labels.shape
    b = indexes.shape[0]
    info = plsc.get_sparse_core_info()
    nw = info.num_cores * info.num_subcores
    b_per_w = b // nw
    mesh = plsc.VectorSubcoreMesh(core_axis_name="c", subcore_axis_name="s")

    @functools.partial(
        pl.kernel,
        mesh=mesh,
        out_type=jax.ShapeDtypeStruct((b, d), jnp.float32),
        scratch_types=[
            pltpu.VMEM((b_per_w,), jnp.int32),
            pltpu.VMEM((b_per_w, d), jnp.float32),
            pltpu.SemaphoreType.DMA,
        ],
    )
    def gather_kernel(table_hbm, idx_hbm, out_hbm, idx_v, rows_v, sem):
        wid = lax.axis_index("s") * info.num_cores + lax.axis_index("c")
        base = wid * b_per_w
        pltpu.sync_copy(idx_hbm.at[pl.ds(base, b_per_w)], idx_v)
        pltpu.async_copy(table_hbm.at[idx_v], rows_v, sem).wait()
        pltpu.sync_copy(rows_v, out_hbm.at[pl.ds(base, b_per_w)])

    return gather_kernel(labels, indexes)


def _nca_tc(embed_sim, idx2d, labels, gathered):
    b, n = embed_sim.shape
    nr, nc = b // _BR, n // _BC
    inv_b = -1.0 / b
    inv_c = 1.0 / _C

    def body(x_ref, idx_ref, lab_ref, g_ref, out_ref, m_acc, z_acc, loss_acc):
        i = pl.program_id(0)
        j = pl.program_id(1)

        @pl.when(j == 0)
        def _():
            m_acc[...] = jnp.zeros_like(m_acc)
            z_acc[...] = jnp.zeros_like(z_acc)

        @pl.when((i == 0) & (j == 0))
        def _():
            loss_acc[0] = 0.0

        idx = idx_ref[...]
        cols = lax.broadcasted_iota(jnp.int32, (_BR, _BC), 1)
        e = jnp.exp(x_ref[...])
        e = jnp.where(cols == (idx - j * _BC), 0.0, e)
        m_acc[...] += jnp.dot(e, lab_ref[...], preferred_element_type=jnp.float32)
        z_acc[...] += jnp.sum(e, axis=1, keepdims=True)

        @pl.when(j == nc - 1)
        def _():
            p = jnp.sum(m_acc[...] * g_ref[...], axis=1, keepdims=True) * inv_c
            prob = p / z_acc[...]
            ll = jnp.log(jnp.where(prob != 0.0, prob, 1.0))
            loss_acc[0] += jnp.sum(ll)

        @pl.when((i == nr - 1) & (j == nc - 1))
        def _():
            out_ref[0, 0] = loss_acc[0] * inv_b

    return pl.pallas_call(
        body,
        grid=(nr, nc),
        in_specs=[
            pl.BlockSpec((_BR, _BC), lambda i, j: (i, j)),
            pl.BlockSpec((_BR, 1), lambda i, j: (i, 0)),
            pl.BlockSpec((_BC, _CP), lambda i, j: (j, 0)),
            pl.BlockSpec((_BR, _CP), lambda i, j: (i, 0)),
        ],
        out_specs=pl.BlockSpec(memory_space=pltpu.SMEM),
        out_shape=jax.ShapeDtypeStruct((1, 1), jnp.float32),
        scratch_shapes=[
            pltpu.VMEM((_BR, _CP), jnp.float32),
            pltpu.VMEM((_BR, 1), jnp.float32),
            pltpu.SMEM((1,), jnp.float32),
        ],
        compiler_params=pltpu.CompilerParams(
            dimension_semantics=("arbitrary", "arbitrary"),
        ),
    )(embed_sim, idx2d, labels, gathered)


def kernel(embed_sim, indexes, labels):
    b, _ = embed_sim.shape
    labels_p = jnp.pad(labels, ((0, 0), (0, _CP - _C)))
    g = _gather_rows_sc(labels_p, indexes)
    out = _nca_tc(embed_sim, indexes.reshape(b, 1), labels_p, g)
    return out[0, 0]
